# Initial kernel scaffold; baseline (speedup 1.0000x reference)
import functools

import jax
import jax.numpy as jnp
from jax import lax
from jax.experimental import pallas as pl
from jax.experimental.pallas import tpu as pltpu
from jax.experimental.pallas import tpu_sc as plsc

N = 10000
E = 320000
D = 128
NC = 2
NS = 16
NW = NC * NS
EPW = E // NW
CHUNK = 80
NCHUNK = EPW // CHUNK
ROWS_PER_TILE = N // NS
ZROWS = 125

_MESH = plsc.VectorSubcoreMesh(
    core_axis_name="c", subcore_axis_name="s", num_cores=NC, num_subcores=NS)


def _relu_add_rows(g_ref, e_ref, nrows):
    def row(r, carry):
        for k in range(D // 16):
            sl = pl.ds(k * 16, 16)
            g_ref[r, sl] = jnp.maximum(g_ref[r, sl] + e_ref[r, sl], 0.0)
        return carry
    lax.fori_loop(0, nrows, row, 0)


@functools.partial(
    pl.kernel,
    out_type=jax.ShapeDtypeStruct((NC, N, D), jnp.float32),
    mesh=_MESH,
    scratch_types=[
        pltpu.VMEM_SHARED((N, D), jnp.float32),
        pltpu.VMEM((CHUNK,), jnp.int32),
        pltpu.VMEM((CHUNK,), jnp.int32),
        pltpu.VMEM((CHUNK, D), jnp.float32),
        pltpu.VMEM((CHUNK, D), jnp.float32),
        pltpu.VMEM((ZROWS, D), jnp.float32),
        pltpu.SemaphoreType.DMA,
    ],
)
def _sc_message(nf_hbm, ef_hbm, src_hbm, dst_hbm, out_hbm,
                acc, idx_dst, idx_src, g, e, zbuf, sem):
    c = lax.axis_index("c")
    s = lax.axis_index("s")
    wid = s * NC + c
    base_w = wid * EPW

    def zrow(r, carry):
        for k in range(D // 16):
            zbuf[r, pl.ds(k * 16, 16)] = jnp.zeros((16,), jnp.float32)
        return carry
    lax.fori_loop(0, ZROWS, zrow, 0)
    for j in range(ROWS_PER_TILE // ZROWS):
        pltpu.sync_copy(zbuf, acc.at[pl.ds(s * ROWS_PER_TILE + j * ZROWS, ZROWS)])
    plsc.subcore_barrier()

    def chunk(i, carry):
        base = base_w + i * CHUNK
        pltpu.sync_copy(dst_hbm.at[pl.ds(base, CHUNK)], idx_dst)
        pltpu.sync_copy(src_hbm.at[pl.ds(base, CHUNK)], idx_src)
        cp = pltpu.async_copy(nf_hbm.at[idx_dst], g, sem)
        pltpu.sync_copy(ef_hbm.at[pl.ds(base, CHUNK)], e)
        cp.wait()
        _relu_add_rows(g, e, CHUNK)
        pltpu.sync_copy(g, acc.at[idx_src], add=True)
        return carry
    lax.fori_loop(0, NCHUNK, chunk, 0)

    plsc.subcore_barrier()
    for j in range(ROWS_PER_TILE // ZROWS):
        sl = pl.ds(s * ROWS_PER_TILE + j * ZROWS, ZROWS)
        pltpu.sync_copy(acc.at[sl], out_hbm.at[c, sl])


@functools.partial(
    pl.kernel,
    out_type=jax.ShapeDtypeStruct((E, 2 * D), jnp.float32),
    mesh=_MESH,
    scratch_types=[
        pltpu.VMEM((CHUNK,), jnp.int32),
        pltpu.VMEM((CHUNK,), jnp.int32),
        pltpu.VMEM((CHUNK, D), jnp.float32),
        pltpu.VMEM((CHUNK, D), jnp.float32),
        pltpu.VMEM((CHUNK, 2 * D), jnp.float32),
        pltpu.SemaphoreType.DMA,
        pltpu.SemaphoreType.DMA,
    ],
)
def _sc_edge_gather(nf_hbm, src_hbm, dst_hbm, out_hbm,
                    idx_src, idx_dst, gi, gj, sa, sem1, sem2):
    c = lax.axis_index("c")
    s = lax.axis_index("s")
    wid = s * NC + c
    base_w = wid * EPW

    def chunk(i, carry):
        base = base_w + i * CHUNK
        pltpu.sync_copy(src_hbm.at[pl.ds(base, CHUNK)], idx_src)
        pltpu.sync_copy(dst_hbm.at[pl.ds(base, CHUNK)], idx_dst)
        cp1 = pltpu.async_copy(nf_hbm.at[idx_src], gi, sem1)
        cp2 = pltpu.async_copy(nf_hbm.at[idx_dst], gj, sem2)
        cp1.wait()
        cp2.wait()

        def row(r, carry2):
            for k in range(D // 16):
                sl = pl.ds(k * 16, 16)
                vi = gi[r, sl]
                vj = gj[r, sl]
                sa[r, pl.ds(k * 16, 16)] = vi + vj
                sa[r, pl.ds(D + k * 16, 16)] = jnp.abs(vi - vj)
            return carry2
        lax.fori_loop(0, CHUNK, row, 0)
        pltpu.sync_copy(sa, out_hbm.at[pl.ds(base, CHUNK)])
        return carry
    lax.fori_loop(0, NCHUNK, chunk, 0)


def _ln(x, g, b):
    m = jnp.mean(x, axis=-1, keepdims=True)
    xc = x - m
    v = jnp.mean(xc * xc, axis=-1, keepdims=True)
    return xc * jax.lax.rsqrt(v + 1e-5) * g + b


BN = 1000


def _node_mlp_body(nf, p0, p1, eps, w1, b1, lg1, lb1, w2, b2, bng, bnb,
                   out_ln, out_relu):
    h = (1.0 + eps[0, 0]) * nf[...] + p0[...] + p1[...]
    t = jnp.dot(h, w1[...], preferred_element_type=jnp.float32) + b1[...]
    t = _ln(t, lg1[...], lb1[...])
    t = jnp.maximum(t, 0.0)
    u = jnp.dot(t, w2[...], preferred_element_type=jnp.float32) + b2[...]
    y = _ln(u, bng[...], bnb[...])
    out_ln[...] = y
    out_relu[...] = jnp.maximum(y, 0.0)


def _tc_node_mlp(nf, p0, p1, eps, w1, b1, lg1, lb1, w2, b2, bng, bnb):
    grid = (N // BN,)
    row_spec = pl.BlockSpec((BN, D), lambda i: (i, 0))
    full = lambda shape: pl.BlockSpec(shape, lambda i: (0, 0))
    return pl.pallas_call(
        _node_mlp_body,
        grid=grid,
        in_specs=[
            row_spec, row_spec, row_spec,
            full((1, 1)),
            full((D, 2 * D)), full((1, 2 * D)), full((1, 2 * D)), full((1, 2 * D)),
            full((2 * D, D)), full((1, D)), full((1, D)), full((1, D)),
        ],
        out_specs=[row_spec, row_spec],
        out_shape=[
            jax.ShapeDtypeStruct((N, D), jnp.float32),
            jax.ShapeDtypeStruct((N, D), jnp.float32),
        ],
    )(nf, p0, p1, eps, w1, b1, lg1, lb1, w2, b2, bng, bnb)


BM = 512


def _edge_mlp_body(sa, ef, wa, wb, wc, b1, lg, lb, w2, b2, out):
    s = sa[:, :D]
    a = sa[:, D:]
    t = jnp.dot(s, wa[...], preferred_element_type=jnp.float32)
    t += jnp.dot(a, wb[...], preferred_element_type=jnp.float32)
    t += jnp.dot(ef[...], wc[...], preferred_element_type=jnp.float32)
    t += b1[...]
    t = _ln(t, lg[...], lb[...])
    t = jnp.maximum(t, 0.0)
    u = jnp.dot(t, w2[...], preferred_element_type=jnp.float32) + b2[...]
    out[...] = u + ef[...]


def _tc_edge_mlp(sa, ef, wa, wb, wc, b1, lg, lb, w2, b2):
    grid = (E // BM,)
    full = lambda shape: pl.BlockSpec(shape, lambda i: (0, 0))
    return pl.pallas_call(
        _edge_mlp_body,
        grid=grid,
        in_specs=[
            pl.BlockSpec((BM, 2 * D), lambda i: (i, 0)),
            pl.BlockSpec((BM, D), lambda i: (i, 0)),
            full((D, 3 * D)), full((D, 3 * D)), full((D, 3 * D)),
            full((1, 3 * D)), full((1, 3 * D)), full((1, 3 * D)),
            full((3 * D, D)), full((1, D)),
        ],
        out_specs=pl.BlockSpec((BM, D), lambda i: (i, 0)),
        out_shape=jax.ShapeDtypeStruct((E, D), jnp.float32),
    )(sa, ef, wa, wb, wc, b1, lg, lb, w2, b2)


def kernel(node_feats, edge_feats, edge_index, params):
    src = edge_index[0]
    dst = edge_index[1]
    nf = node_feats
    ef = edge_feats
    nf_ln = node_feats
    num_layers = len(params)
    for l in range(num_layers):
        p = params["layer%d" % l]
        parts = _sc_message(nf, ef, src, dst)
        nf_ln, nf_relu = _tc_node_mlp(
            nf, parts[0], parts[1],
            p["eps"].reshape(1, 1),
            p["cW1"], p["cb1"].reshape(1, -1),
            p["cln_g"].reshape(1, -1), p["cln_b"].reshape(1, -1),
            p["cW2"], p["cb2"].reshape(1, -1),
            p["bn_g"].reshape(1, -1), p["bn_b"].reshape(1, -1),
        )
        sa = _sc_edge_gather(nf_ln, src, dst)
        ef = _tc_edge_mlp(
            sa, ef,
            p["eW1"][:D], p["eW1"][D:2 * D], p["eW1"][2 * D:],
            p["eb1"].reshape(1, -1),
            p["eln_g"].reshape(1, -1), p["eln_b"].reshape(1, -1),
            p["eW2"], p["eb2"].reshape(1, -1),
        )
        nf = nf_relu
    return nf_ln, ef


# speedup vs baseline: 1.7482x; 1.7482x over previous
import functools

import jax
import jax.numpy as jnp
from jax import lax
from jax.experimental import pallas as pl
from jax.experimental.pallas import tpu as pltpu
from jax.experimental.pallas import tpu_sc as plsc

N = 10000
E = 320000
D = 128
NC = 2
NS = 16
NW = NC * NS
EPW = E // NW
CHUNK = 80
NCHUNK = EPW // CHUNK
ZROWS = 624
REM_BASE = NS * ZROWS
REM = N - REM_BASE
ZBUF = 48

_MESH = plsc.VectorSubcoreMesh(
    core_axis_name="c", subcore_axis_name="s", num_cores=NC, num_subcores=NS)


def _relu_add_rows(g_ref, e_ref, nrows):
    def row(r, carry):
        for k in range(D // 16):
            sl = pl.ds(k * 16, 16)
            g_ref[r, sl] = jnp.maximum(g_ref[r, sl] + e_ref[r, sl], 0.0)
        return carry
    lax.fori_loop(0, nrows, row, 0)


@functools.partial(
    pl.kernel,
    out_type=jax.ShapeDtypeStruct((NC, N, D), jnp.float32),
    mesh=_MESH,
    scratch_types=[
        pltpu.VMEM_SHARED((N, D), jnp.float32),
        pltpu.VMEM((CHUNK,), jnp.int32),
        pltpu.VMEM((CHUNK,), jnp.int32),
        pltpu.VMEM((CHUNK, D), jnp.float32),
        pltpu.VMEM((CHUNK, D), jnp.float32),
        pltpu.VMEM((ZBUF, D), jnp.float32),
        pltpu.SemaphoreType.DMA,
    ],
)
def _sc_message(nf_hbm, ef_hbm, src_hbm, dst_hbm, out_hbm,
                acc, idx_dst, idx_src, g, e, zbuf, sem):
    c = lax.axis_index("c")
    s = lax.axis_index("s")
    wid = s * NC + c
    base_w = wid * EPW

    def zrow(r, carry):
        for k in range(D // 16):
            zbuf[r, pl.ds(k * 16, 16)] = jnp.zeros((16,), jnp.float32)
        return carry
    lax.fori_loop(0, ZBUF, zrow, 0)
    for j in range(ZROWS // ZBUF):
        pltpu.sync_copy(zbuf, acc.at[pl.ds(s * ZROWS + j * ZBUF, ZBUF)])

    @pl.when(s == NS - 1)
    def _():
        pltpu.sync_copy(zbuf.at[pl.ds(0, REM)], acc.at[pl.ds(REM_BASE, REM)])
    plsc.subcore_barrier()

    def chunk(i, carry):
        base = base_w + i * CHUNK
        pltpu.sync_copy(dst_hbm.at[pl.ds(base, CHUNK)], idx_dst)
        pltpu.sync_copy(src_hbm.at[pl.ds(base, CHUNK)], idx_src)
        cp = pltpu.async_copy(nf_hbm.at[idx_dst], g, sem)
        pltpu.sync_copy(ef_hbm.at[pl.ds(base, CHUNK)], e)
        cp.wait()
        _relu_add_rows(g, e, CHUNK)
        pltpu.sync_copy(g, acc.at[idx_src], add=True)
        return carry
    lax.fori_loop(0, NCHUNK, chunk, 0)

    plsc.subcore_barrier()
    sl = pl.ds(s * ZROWS, ZROWS)
    pltpu.sync_copy(acc.at[sl], out_hbm.at[c, sl])

    @pl.when(s == NS - 1)
    def _():
        rsl = pl.ds(REM_BASE, REM)
        pltpu.sync_copy(acc.at[rsl], out_hbm.at[c, rsl])


@functools.partial(
    pl.kernel,
    out_type=jax.ShapeDtypeStruct((E, 2 * D), jnp.float32),
    mesh=_MESH,
    scratch_types=[
        pltpu.VMEM((CHUNK,), jnp.int32),
        pltpu.VMEM((CHUNK,), jnp.int32),
        pltpu.VMEM((CHUNK, D), jnp.float32),
        pltpu.VMEM((CHUNK, D), jnp.float32),
        pltpu.VMEM((CHUNK, 2 * D), jnp.float32),
        pltpu.SemaphoreType.DMA,
        pltpu.SemaphoreType.DMA,
    ],
)
def _sc_edge_gather(nf_hbm, src_hbm, dst_hbm, out_hbm,
                    idx_src, idx_dst, gi, gj, sa, sem1, sem2):
    c = lax.axis_index("c")
    s = lax.axis_index("s")
    wid = s * NC + c
    base_w = wid * EPW

    def chunk(i, carry):
        base = base_w + i * CHUNK
        pltpu.sync_copy(src_hbm.at[pl.ds(base, CHUNK)], idx_src)
        pltpu.sync_copy(dst_hbm.at[pl.ds(base, CHUNK)], idx_dst)
        cp1 = pltpu.async_copy(nf_hbm.at[idx_src], gi, sem1)
        cp2 = pltpu.async_copy(nf_hbm.at[idx_dst], gj, sem2)
        cp1.wait()
        cp2.wait()

        def row(r, carry2):
            for k in range(D // 16):
                sl = pl.ds(k * 16, 16)
                vi = gi[r, sl]
                vj = gj[r, sl]
                sa[r, pl.ds(k * 16, 16)] = vi + vj
                sa[r, pl.ds(D + k * 16, 16)] = jnp.abs(vi - vj)
            return carry2
        lax.fori_loop(0, CHUNK, row, 0)
        pltpu.sync_copy(sa, out_hbm.at[pl.ds(base, CHUNK)])
        return carry
    lax.fori_loop(0, NCHUNK, chunk, 0)


def _ln(x, g, b):
    m = jnp.mean(x, axis=-1, keepdims=True)
    xc = x - m
    v = jnp.mean(xc * xc, axis=-1, keepdims=True)
    return xc * jax.lax.rsqrt(v + 1e-5) * g + b


BN = 1000


def _node_mlp_body(nf, p0, p1, eps, w1, b1, lg1, lb1, w2, b2, bng, bnb,
                   out_ln, out_relu):
    h = (1.0 + eps[0, 0]) * nf[...] + p0[...] + p1[...]
    t = jnp.dot(h, w1[...], preferred_element_type=jnp.float32) + b1[...]
    t = _ln(t, lg1[...], lb1[...])
    t = jnp.maximum(t, 0.0)
    u = jnp.dot(t, w2[...], preferred_element_type=jnp.float32) + b2[...]
    y = _ln(u, bng[...], bnb[...])
    out_ln[...] = y
    out_relu[...] = jnp.maximum(y, 0.0)


def _tc_node_mlp(nf, p0, p1, eps, w1, b1, lg1, lb1, w2, b2, bng, bnb):
    grid = (N // BN,)
    row_spec = pl.BlockSpec((BN, D), lambda i: (i, 0))
    full = lambda shape: pl.BlockSpec(shape, lambda i: (0, 0))
    return pl.pallas_call(
        _node_mlp_body,
        grid=grid,
        in_specs=[
            row_spec, row_spec, row_spec,
            full((1, 1)),
            full((D, 2 * D)), full((1, 2 * D)), full((1, 2 * D)), full((1, 2 * D)),
            full((2 * D, D)), full((1, D)), full((1, D)), full((1, D)),
        ],
        out_specs=[row_spec, row_spec],
        out_shape=[
            jax.ShapeDtypeStruct((N, D), jnp.float32),
            jax.ShapeDtypeStruct((N, D), jnp.float32),
        ],
    )(nf, p0, p1, eps, w1, b1, lg1, lb1, w2, b2, bng, bnb)


BM = 512


def _edge_mlp_body(sa, ef, wa, wb, wc, b1, lg, lb, w2, b2, out):
    s = sa[:, :D]
    a = sa[:, D:]
    t = jnp.dot(s, wa[...], preferred_element_type=jnp.float32)
    t += jnp.dot(a, wb[...], preferred_element_type=jnp.float32)
    t += jnp.dot(ef[...], wc[...], preferred_element_type=jnp.float32)
    t += b1[...]
    t = _ln(t, lg[...], lb[...])
    t = jnp.maximum(t, 0.0)
    u = jnp.dot(t, w2[...], preferred_element_type=jnp.float32) + b2[...]
    out[...] = u + ef[...]


def _tc_edge_mlp(sa, ef, wa, wb, wc, b1, lg, lb, w2, b2):
    grid = (E // BM,)
    full = lambda shape: pl.BlockSpec(shape, lambda i: (0, 0))
    return pl.pallas_call(
        _edge_mlp_body,
        grid=grid,
        in_specs=[
            pl.BlockSpec((BM, 2 * D), lambda i: (i, 0)),
            pl.BlockSpec((BM, D), lambda i: (i, 0)),
            full((D, 3 * D)), full((D, 3 * D)), full((D, 3 * D)),
            full((1, 3 * D)), full((1, 3 * D)), full((1, 3 * D)),
            full((3 * D, D)), full((1, D)),
        ],
        out_specs=pl.BlockSpec((BM, D), lambda i: (i, 0)),
        out_shape=jax.ShapeDtypeStruct((E, D), jnp.float32),
    )(sa, ef, wa, wb, wc, b1, lg, lb, w2, b2)


def kernel(node_feats, edge_feats, edge_index, params):
    src = edge_index[0]
    dst = edge_index[1]
    nf = node_feats
    ef = edge_feats
    nf_ln = node_feats
    num_layers = len(params)
    for l in range(num_layers):
        p = params["layer%d" % l]
        parts = _sc_message(nf, ef, src, dst)
        nf_ln, nf_relu = _tc_node_mlp(
            nf, parts[0], parts[1],
            p["eps"].reshape(1, 1),
            p["cW1"], p["cb1"].reshape(1, -1),
            p["cln_g"].reshape(1, -1), p["cln_b"].reshape(1, -1),
            p["cW2"], p["cb2"].reshape(1, -1),
            p["bn_g"].reshape(1, -1), p["bn_b"].reshape(1, -1),
        )
        sa = _sc_edge_gather(nf_ln, src, dst)
        ef = _tc_edge_mlp(
            sa, ef,
            p["eW1"][:D], p["eW1"][D:2 * D], p["eW1"][2 * D:],
            p["eb1"].reshape(1, -1),
            p["eln_g"].reshape(1, -1), p["eln_b"].reshape(1, -1),
            p["eW2"], p["eb2"].reshape(1, -1),
        )
        nf = nf_relu
    return nf_ln, ef


# speedup vs baseline: 2.2481x; 1.2860x over previous
import functools

import jax
import jax.numpy as jnp
from jax import lax
from jax.experimental import pallas as pl
from jax.experimental.pallas import tpu as pltpu
from jax.experimental.pallas import tpu_sc as plsc

N = 10000
E = 320000
D = 128
NC = 2
NS = 16
NW = NC * NS
EPW = E // NW
CHUNK = 80
NCHUNK = EPW // CHUNK
ZROWS = 624
REM_BASE = NS * ZROWS
REM = N - REM_BASE
ZBUF = 16

NBUF_G = 5
GGRP = NBUF_G * CHUNK
NGRP_G = EPW // GGRP

NBUF_M = 2
MGRP = NBUF_M * CHUNK
NGRP_M = NCHUNK // NBUF_M

_MESH = plsc.VectorSubcoreMesh(
    core_axis_name="c", subcore_axis_name="s", num_cores=NC, num_subcores=NS)


def _relu_add_rows(g_ref, e_ref):
    def row(r, carry):
        for k in range(D // 16):
            sl = pl.ds(k * 16, 16)
            g_ref[r, sl] = jnp.maximum(g_ref[r, sl] + e_ref[r, sl], 0.0)
        return carry
    lax.fori_loop(0, CHUNK, row, 0, unroll=2)


_MSG_SCRATCH = (
    [pltpu.VMEM_SHARED((N, D), jnp.float32)]
    + [pltpu.VMEM((MGRP,), jnp.int32)]
    + [pltpu.VMEM((CHUNK,), jnp.int32) for _ in range(NBUF_M)]
    + [pltpu.VMEM((CHUNK, D), jnp.float32) for _ in range(NBUF_M)]
    + [pltpu.VMEM((CHUNK, D), jnp.float32) for _ in range(NBUF_M)]
    + [pltpu.VMEM((ZBUF, D), jnp.float32)]
    + [pltpu.SemaphoreType.DMA for _ in range(4 * NBUF_M)]
)


@functools.partial(
    pl.kernel,
    out_type=jax.ShapeDtypeStruct((NC, N, D), jnp.float32),
    mesh=_MESH,
    scratch_types=_MSG_SCRATCH,
)
def _sc_message(nf_hbm, ef_hbm, src_hbm, dst_hbm, out_hbm, *sc):
    acc = sc[0]
    gd = sc[1]
    ss = sc[2:2 + NBUF_M]
    gb = sc[4:4 + NBUF_M]
    eb = sc[6:6 + NBUF_M]
    zbuf = sc[8]
    isem = sc[9:9 + NBUF_M]
    gsem = sc[11:11 + NBUF_M]
    esem = sc[13:13 + NBUF_M]
    ssem = sc[15:15 + NBUF_M]

    c = lax.axis_index("c")
    s = lax.axis_index("s")
    wid = s * NC + c
    base_w = wid * EPW

    def zrow(r, carry):
        for k in range(D // 16):
            zbuf[r, pl.ds(k * 16, 16)] = jnp.zeros((16,), jnp.float32)
        return carry
    lax.fori_loop(0, ZBUF, zrow, 0)
    for j in range(ZROWS // ZBUF):
        pltpu.sync_copy(zbuf, acc.at[pl.ds(s * ZROWS + j * ZBUF, ZBUF)])

    @pl.when(s == NS - 1)
    def _():
        pltpu.sync_copy(zbuf.at[pl.ds(0, REM)], acc.at[pl.ds(REM_BASE, REM)])
    plsc.subcore_barrier()

    def mgroup(base_g, first):
        pltpu.sync_copy(dst_hbm.at[pl.ds(base_g, MGRP)], gd)
        hs = []
        for b in range(NBUF_M):
            base = base_g + b * CHUNK
            if not first:
                pltpu.make_async_copy(gb[b], acc.at[pl.ds(0, CHUNK)],
                                      ssem[b]).wait()
            hi = pltpu.async_copy(src_hbm.at[pl.ds(base, CHUNK)], ss[b], isem[b])
            hg = pltpu.async_copy(nf_hbm.at[gd.at[pl.ds(b * CHUNK, CHUNK)]],
                                  gb[b], gsem[b])
            he = pltpu.async_copy(ef_hbm.at[pl.ds(base, CHUNK)], eb[b], esem[b])
            hs.append((hi, hg, he))
        for b in range(NBUF_M):
            hi, hg, he = hs[b]
            hg.wait()
            he.wait()
            _relu_add_rows(gb[b], eb[b])
            hi.wait()
            pltpu.async_copy(gb[b], acc.at[ss[b]], ssem[b], add=True)

    mgroup(base_w, True)
    lax.fori_loop(1, NGRP_M,
                  lambda g, carry: (mgroup(base_w + g * MGRP, False), carry)[1],
                  0)

    tbase = base_w + NGRP_M * MGRP
    pltpu.make_async_copy(gb[0], acc.at[pl.ds(0, CHUNK)], ssem[0]).wait()
    pltpu.sync_copy(dst_hbm.at[pl.ds(tbase, CHUNK)], gd.at[pl.ds(0, CHUNK)])
    hi = pltpu.async_copy(src_hbm.at[pl.ds(tbase, CHUNK)], ss[0], isem[0])
    hg = pltpu.async_copy(nf_hbm.at[gd.at[pl.ds(0, CHUNK)]], gb[0], gsem[0])
    he = pltpu.async_copy(ef_hbm.at[pl.ds(tbase, CHUNK)], eb[0], esem[0])
    hg.wait()
    he.wait()
    _relu_add_rows(gb[0], eb[0])
    hi.wait()
    pltpu.async_copy(gb[0], acc.at[ss[0]], ssem[0], add=True)
    for b in range(NBUF_M):
        pltpu.make_async_copy(gb[b], acc.at[pl.ds(0, CHUNK)], ssem[b]).wait()

    plsc.subcore_barrier()
    sl = pl.ds(s * ZROWS, ZROWS)
    pltpu.sync_copy(acc.at[sl], out_hbm.at[c, sl])

    @pl.when(s == NS - 1)
    def _():
        rsl = pl.ds(REM_BASE, REM)
        pltpu.sync_copy(acc.at[rsl], out_hbm.at[c, rsl])


_GATHER_SCRATCH = (
    [pltpu.VMEM((GGRP,), jnp.int32) for _ in range(2)]
    + [pltpu.VMEM((CHUNK, D), jnp.float32) for _ in range(2 * NBUF_G)]
    + [pltpu.SemaphoreType.DMA for _ in range(4 * NBUF_G)]
)


@functools.partial(
    pl.kernel,
    out_type=(
        jax.ShapeDtypeStruct((E, D), jnp.float32),
        jax.ShapeDtypeStruct((E, D), jnp.float32),
    ),
    mesh=_MESH,
    scratch_types=_GATHER_SCRATCH,
)
def _sc_edge_gather(nf_hbm, src_hbm, dst_hbm, gi_hbm, gj_hbm, *sc):
    isrc = sc[0]
    idst = sc[1]
    gbi = sc[2:2 + NBUF_G]
    gbj = sc[7:7 + NBUF_G]
    gsi = sc[12:12 + NBUF_G]
    gsj = sc[17:17 + NBUF_G]
    wsi = sc[22:22 + NBUF_G]
    wsj = sc[27:27 + NBUF_G]

    c = lax.axis_index("c")
    s = lax.axis_index("s")
    wid = s * NC + c
    base_w = wid * EPW

    def group(base_g, first):
        pltpu.sync_copy(src_hbm.at[pl.ds(base_g, GGRP)], isrc)
        pltpu.sync_copy(dst_hbm.at[pl.ds(base_g, GGRP)], idst)
        hs = []
        for b in range(NBUF_G):
            if not first:
                pltpu.make_async_copy(gbi[b], gi_hbm.at[pl.ds(0, CHUNK)],
                                      wsi[b]).wait()
                pltpu.make_async_copy(gbj[b], gj_hbm.at[pl.ds(0, CHUNK)],
                                      wsj[b]).wait()
            h1 = pltpu.async_copy(nf_hbm.at[isrc.at[pl.ds(b * CHUNK, CHUNK)]],
                                  gbi[b], gsi[b])
            h2 = pltpu.async_copy(nf_hbm.at[idst.at[pl.ds(b * CHUNK, CHUNK)]],
                                  gbj[b], gsj[b])
            hs.append((h1, h2))
        for b in range(NBUF_G):
            h1, h2 = hs[b]
            sl = pl.ds(base_g + b * CHUNK, CHUNK)
            h1.wait()
            pltpu.async_copy(gbi[b], gi_hbm.at[sl], wsi[b])
            h2.wait()
            pltpu.async_copy(gbj[b], gj_hbm.at[sl], wsj[b])

    group(base_w, True)
    lax.fori_loop(1, NGRP_G,
                  lambda g, carry: (group(base_w + g * GGRP, False), carry)[1],
                  0)
    for b in range(NBUF_G):
        pltpu.make_async_copy(gbi[b], gi_hbm.at[pl.ds(0, CHUNK)], wsi[b]).wait()
        pltpu.make_async_copy(gbj[b], gj_hbm.at[pl.ds(0, CHUNK)], wsj[b]).wait()


def _ln(x, g, b):
    m = jnp.mean(x, axis=-1, keepdims=True)
    xc = x - m
    v = jnp.mean(xc * xc, axis=-1, keepdims=True)
    return xc * jax.lax.rsqrt(v + 1e-5) * g + b


BN = 1000


def _node_mlp_body(nf, p0, p1, eps, w1, b1, lg1, lb1, w2, b2, bng, bnb,
                   out_ln, out_relu):
    h = (1.0 + eps[0, 0]) * nf[...] + p0[...] + p1[...]
    t = jnp.dot(h, w1[...], preferred_element_type=jnp.float32) + b1[...]
    t = _ln(t, lg1[...], lb1[...])
    t = jnp.maximum(t, 0.0)
    u = jnp.dot(t, w2[...], preferred_element_type=jnp.float32) + b2[...]
    y = _ln(u, bng[...], bnb[...])
    out_ln[...] = y
    out_relu[...] = jnp.maximum(y, 0.0)


def _tc_node_mlp(nf, p0, p1, eps, w1, b1, lg1, lb1, w2, b2, bng, bnb):
    grid = (N // BN,)
    row_spec = pl.BlockSpec((BN, D), lambda i: (i, 0))
    full = lambda shape: pl.BlockSpec(shape, lambda i: (0, 0))
    return pl.pallas_call(
        _node_mlp_body,
        grid=grid,
        in_specs=[
            row_spec, row_spec, row_spec,
            full((1, 1)),
            full((D, 2 * D)), full((1, 2 * D)), full((1, 2 * D)), full((1, 2 * D)),
            full((2 * D, D)), full((1, D)), full((1, D)), full((1, D)),
        ],
        out_specs=[row_spec, row_spec],
        out_shape=[
            jax.ShapeDtypeStruct((N, D), jnp.float32),
            jax.ShapeDtypeStruct((N, D), jnp.float32),
        ],
    )(nf, p0, p1, eps, w1, b1, lg1, lb1, w2, b2, bng, bnb)


BM = 512


def _edge_mlp_body(gi, gj, ef, wa, wb, wc, b1, lg, lb, w2, b2, out):
    s = gi[...] + gj[...]
    a = jnp.abs(gi[...] - gj[...])
    t = jnp.dot(s, wa[...], preferred_element_type=jnp.float32)
    t += jnp.dot(a, wb[...], preferred_element_type=jnp.float32)
    t += jnp.dot(ef[...], wc[...], preferred_element_type=jnp.float32)
    t += b1[...]
    t = _ln(t, lg[...], lb[...])
    t = jnp.maximum(t, 0.0)
    u = jnp.dot(t, w2[...], preferred_element_type=jnp.float32) + b2[...]
    out[...] = u + ef[...]


def _tc_edge_mlp(gi, gj, ef, wa, wb, wc, b1, lg, lb, w2, b2):
    grid = (E // BM,)
    full = lambda shape: pl.BlockSpec(shape, lambda i: (0, 0))
    row_spec = pl.BlockSpec((BM, D), lambda i: (i, 0))
    return pl.pallas_call(
        _edge_mlp_body,
        grid=grid,
        in_specs=[
            row_spec, row_spec, row_spec,
            full((D, 3 * D)), full((D, 3 * D)), full((D, 3 * D)),
            full((1, 3 * D)), full((1, 3 * D)), full((1, 3 * D)),
            full((3 * D, D)), full((1, D)),
        ],
        out_specs=row_spec,
        out_shape=jax.ShapeDtypeStruct((E, D), jnp.float32),
    )(gi, gj, ef, wa, wb, wc, b1, lg, lb, w2, b2)


def kernel(node_feats, edge_feats, edge_index, params):
    src = edge_index[0]
    dst = edge_index[1]
    nf = node_feats
    ef = edge_feats
    nf_ln = node_feats
    num_layers = len(params)
    for l in range(num_layers):
        p = params["layer%d" % l]
        parts = _sc_message(nf, ef, src, dst)
        nf_ln, nf_relu = _tc_node_mlp(
            nf, parts[0], parts[1],
            p["eps"].reshape(1, 1),
            p["cW1"], p["cb1"].reshape(1, -1),
            p["cln_g"].reshape(1, -1), p["cln_b"].reshape(1, -1),
            p["cW2"], p["cb2"].reshape(1, -1),
            p["bn_g"].reshape(1, -1), p["bn_b"].reshape(1, -1),
        )
        gi, gj = _sc_edge_gather(nf_ln, src, dst)
        ef = _tc_edge_mlp(
            gi, gj, ef,
            p["eW1"][:D], p["eW1"][D:2 * D], p["eW1"][2 * D:],
            p["eb1"].reshape(1, -1),
            p["eln_g"].reshape(1, -1), p["eln_b"].reshape(1, -1),
            p["eW2"], p["eb2"].reshape(1, -1),
        )
        nf = nf_relu
    return nf_ln, ef
